# baseline (speedup 1.0000x reference)
import functools

import jax
import jax.numpy as jnp
from jax import lax
from jax.experimental import pallas as pl
from jax.experimental.pallas import tpu as pltpu
from jax.experimental.pallas import tpu_sc as plsc

N = 10000
E = 320000
D = 128

NC = 2
NS = 16
NW = NC * NS

CHUNK = 128
KPT = 79
NCHUNKS = NW * KPT
E_PAD = NCHUNKS * CHUNK
NPAD = N + 16

ROWS_T = 624
TAIL = NPAD - 15 * ROWS_T


def _sc_body(feat_hbm, src_hbm, dst_hbm, sums_out, cnts_out,
             sums_sh, cnts_sh, srcv0, dstv0, srcv1, dstv1,
             rows0, rows1, onesv, zcnt, sem0, sem1):
    cid = lax.axis_index("c")
    sid = lax.axis_index("s")
    wid = sid * NC + cid

    zeros16 = jnp.zeros((16,), jnp.float32)
    ones16 = jnp.ones((16,), jnp.float32)

    def zr_row(r, _):
        def zr_col(j, _):
            rows0[r, pl.ds(pl.multiple_of(j * 16, 16), 16)] = zeros16
            return 0
        return lax.fori_loop(0, D // 16, zr_col, 0)

    lax.fori_loop(0, CHUNK, zr_row, 0)

    def zc(i, _):
        zcnt[pl.ds(pl.multiple_of(i * 16, 16), 16)] = zeros16
        return 0

    lax.fori_loop(0, TAIL // 16, zc, 0)

    for j in range(CHUNK // 16):
        onesv[pl.ds(j * 16, 16)] = ones16

    row0 = sid * ROWS_T
    for b in range(4):
        pltpu.sync_copy(rows0, sums_sh.at[pl.ds(row0 + b * CHUNK, CHUNK)])
    pltpu.sync_copy(rows0.at[pl.ds(0, 112)],
                    sums_sh.at[pl.ds(row0 + 4 * CHUNK, 112)])
    pltpu.sync_copy(zcnt.at[pl.ds(0, ROWS_T)], cnts_sh.at[pl.ds(row0, ROWS_T)])

    @pl.when(sid == NS - 1)
    def _():
        pltpu.sync_copy(rows0.at[pl.ds(0, 32)],
                        sums_sh.at[pl.ds(NPAD - 32, 32)])
        pltpu.sync_copy(zcnt.at[pl.ds(0, 32)],
                        cnts_sh.at[pl.ds(NPAD - 32, 32)])

    plsc.subcore_barrier()

    chunk0 = wid * KPT

    def load_idx(c, sv, dv):
        base = pl.multiple_of(c * CHUNK, CHUNK)
        pltpu.sync_copy(src_hbm.at[pl.ds(base, CHUNK)], sv)
        pltpu.sync_copy(dst_hbm.at[pl.ds(base, CHUNK)], dv)

    def scat(rows, dv):
        pltpu.sync_copy(rows, sums_sh.at[dv], add=True)
        pltpu.sync_copy(onesv, cnts_sh.at[dv], add=True)

    load_idx(chunk0, srcv0, dstv0)
    g0 = pltpu.async_copy(feat_hbm.at[srcv0], rows0, sem0)

    def pair(i, _):
        c0 = chunk0 + 2 * i
        load_idx(c0 + 1, srcv1, dstv1)
        pltpu.make_async_copy(feat_hbm.at[srcv0], rows0, sem0).wait()
        pltpu.async_copy(feat_hbm.at[srcv1], rows1, sem1)
        scat(rows0, dstv0)
        load_idx(c0 + 2, srcv0, dstv0)
        pltpu.make_async_copy(feat_hbm.at[srcv1], rows1, sem1).wait()
        pltpu.async_copy(feat_hbm.at[srcv0], rows0, sem0)
        scat(rows1, dstv1)
        return 0

    lax.fori_loop(0, (KPT - 1) // 2, pair, 0)

    pltpu.make_async_copy(feat_hbm.at[srcv0], rows0, sem0).wait()
    scat(rows0, dstv0)
    del g0

    plsc.subcore_barrier()

    out0 = cid * N + row0
    for b in range(4):
        buf = rows0 if b % 2 == 0 else rows1
        pltpu.sync_copy(sums_sh.at[pl.ds(row0 + b * CHUNK, CHUNK)], buf)
        pltpu.sync_copy(buf, sums_out.at[pl.ds(out0 + b * CHUNK, CHUNK)])
    pltpu.sync_copy(sums_sh.at[pl.ds(row0 + 4 * CHUNK, 112)],
                    rows0.at[pl.ds(0, 112)])
    pltpu.sync_copy(rows0.at[pl.ds(0, 112)],
                    sums_out.at[pl.ds(out0 + 4 * CHUNK, 112)])
    pltpu.sync_copy(cnts_sh.at[pl.ds(row0, ROWS_T)], zcnt.at[pl.ds(0, ROWS_T)])
    pltpu.sync_copy(zcnt.at[pl.ds(0, ROWS_T)], cnts_out.at[pl.ds(out0, ROWS_T)])

    @pl.when(sid == NS - 1)
    def _():
        pltpu.sync_copy(sums_sh.at[pl.ds(N - 16, 16)], rows1.at[pl.ds(0, 16)])
        pltpu.sync_copy(rows1.at[pl.ds(0, 16)],
                        sums_out.at[pl.ds(cid * N + N - 16, 16)])
        pltpu.sync_copy(cnts_sh.at[pl.ds(N - 16, 16)], zcnt.at[pl.ds(0, 16)])
        pltpu.sync_copy(zcnt.at[pl.ds(0, 16)],
                        cnts_out.at[pl.ds(cid * N + N - 16, 16)])


_sc_scatter = functools.partial(
    pl.kernel,
    out_type=(
        jax.ShapeDtypeStruct((NC * N, D), jnp.float32),
        jax.ShapeDtypeStruct((NC * N,), jnp.float32),
    ),
    mesh=plsc.VectorSubcoreMesh(core_axis_name="c", subcore_axis_name="s"),
    scratch_types=(
        pltpu.VMEM_SHARED((NPAD, D), jnp.float32),
        pltpu.VMEM_SHARED((NPAD,), jnp.float32),
        pltpu.VMEM((CHUNK,), jnp.int32),
        pltpu.VMEM((CHUNK,), jnp.int32),
        pltpu.VMEM((CHUNK,), jnp.int32),
        pltpu.VMEM((CHUNK,), jnp.int32),
        pltpu.VMEM((CHUNK, D), jnp.float32),
        pltpu.VMEM((CHUNK, D), jnp.float32),
        pltpu.VMEM((CHUNK,), jnp.float32),
        pltpu.VMEM((TAIL,), jnp.float32),
        pltpu.SemaphoreType.DMA,
        pltpu.SemaphoreType.DMA,
    ),
)(_sc_body)


BLK = 1000


def _tc_body(f_ref, w_ref, s0_ref, s1_ref, c0_ref, c1_ref, o_ref):
    w = w_ref[...]
    s = s0_ref[...] + s1_ref[...]
    cnt = c0_ref[...] + c1_ref[...]
    mean = s * (1.0 / jnp.maximum(cnt, 1.0))
    nodes = jnp.dot(f_ref[...], w, preferred_element_type=jnp.float32)
    agg = jnp.dot(mean, w, preferred_element_type=jnp.float32)
    o_ref[:, :D] = jnp.maximum(nodes, 0.0)
    o_ref[:, D:] = jnp.maximum(agg, 0.0)


def _tc_dense(features, weight, sums2, cnts2):
    return pl.pallas_call(
        _tc_body,
        grid=(N // BLK,),
        in_specs=[
            pl.BlockSpec((BLK, D), lambda i: (i, 0)),
            pl.BlockSpec((D, D), lambda i: (0, 0)),
            pl.BlockSpec((BLK, D), lambda i: (i, 0)),
            pl.BlockSpec((BLK, D), lambda i: (N // BLK + i, 0)),
            pl.BlockSpec((BLK, 1), lambda i: (i, 0)),
            pl.BlockSpec((BLK, 1), lambda i: (N // BLK + i, 0)),
        ],
        out_specs=pl.BlockSpec((BLK, 2 * D), lambda i: (i, 0)),
        out_shape=jax.ShapeDtypeStruct((N, 2 * D), jnp.float32),
    )(features, weight, sums2, sums2, cnts2, cnts2)


def kernel(features, edges, weight):
    edges = edges.astype(jnp.int32)
    npad = E_PAD - E
    dst = jnp.concatenate([edges[0], jnp.full((npad,), N, jnp.int32)])
    src = jnp.concatenate([edges[1], jnp.zeros((npad,), jnp.int32)])
    sums2, cnts2 = _sc_scatter(features, src, dst)
    return _tc_dense(features, weight, sums2, cnts2.reshape(NC * N, 1))
